# Initial kernel scaffold; baseline (speedup 1.0000x reference)
import jax
import jax.numpy as jnp
from jax.experimental import pallas as pl


def kernel(adjacency, input_feature, W, b_lin, bias_out):
    raise NotImplementedError("write your pallas kernel here")


# speedup vs baseline: 1.9906x; 1.9906x over previous
import jax
import jax.numpy as jnp
from jax.experimental import pallas as pl
from jax.experimental.pallas import tpu as pltpu

_C = 1.0
_EPS = 1e-5
_MIN_NORM = 1e-15


def _artanh(x):
    x = jnp.clip(x, -1 + 1e-7, 1 - 1e-7)
    return 0.5 * jnp.log((1 + x) / (1 - x))


def _row_norm(x):
    return jnp.clip(jnp.sqrt(jnp.sum(x * x, axis=-1, keepdims=True)), _MIN_NORM, None)


def _proj(x):
    norm = _row_norm(x)
    maxnorm = (1 - _EPS)
    return jnp.where(norm > maxnorm, x / norm * maxnorm, x)


def _expmap0(u):
    u_norm = _row_norm(u)
    return jnp.tanh(u_norm) * u / u_norm


def _hgc_kernel(adj_ref, x_ref, w_ref, b_ref, bo_ref, out_ref, xt_ref):
    i = pl.program_id(0)

    @pl.when(i == 0)
    def _stage1():
        x = x_ref[...]
        w = w_ref[...]
        x_norm = _row_norm(x)
        mx = jax.lax.dot_general(
            x, w, (((1,), (1,)), ((), ())), preferred_element_type=jnp.float32
        )
        mx_norm = _row_norm(mx)
        res_c = jnp.tanh(mx_norm / x_norm * _artanh(x_norm)) * mx / mx_norm
        zero_row = jnp.all(mx == 0, axis=-1, keepdims=True)
        res = _proj(jnp.where(zero_row, jnp.zeros_like(res_c), res_c))
        hb = _proj(_expmap0(b_ref[...]))
        x2 = jnp.sum(res * res, axis=-1, keepdims=True)
        y2 = jnp.sum(hb * hb, axis=-1, keepdims=True)
        xy = jnp.sum(res * hb, axis=-1, keepdims=True)
        num = (1 + 2 * xy + y2) * res + (1 - x2) * hb
        den = jnp.clip(1 + 2 * xy + x2 * y2, _MIN_NORM, None)
        res2 = _proj(num / den)
        p_norm = _row_norm(res2)
        xt_ref[...] = res2 / p_norm * _artanh(p_norm)

    @pl.when(i > 0)
    def _stage2():
        s = jnp.dot(adj_ref[...], xt_ref[...], preferred_element_type=jnp.float32)
        out_ref[...] = _proj(_expmap0(s)) + bo_ref[...]


def kernel(adjacency, input_feature, W, b_lin, bias_out):
    N, din = input_feature.shape
    dout = W.shape[0]
    BM = 512
    nblk = N // BM
    b2 = b_lin.reshape(1, dout).astype(jnp.float32)
    bo2 = bias_out.reshape(1, dout).astype(jnp.float32)
    return pl.pallas_call(
        _hgc_kernel,
        grid=(nblk + 1,),
        in_specs=[
            pl.BlockSpec((BM, N), lambda i: (jnp.maximum(i - 1, 0), 0)),
            pl.BlockSpec((N, din), lambda i: (0, 0)),
            pl.BlockSpec((dout, din), lambda i: (0, 0)),
            pl.BlockSpec((1, dout), lambda i: (0, 0)),
            pl.BlockSpec((1, dout), lambda i: (0, 0)),
        ],
        out_specs=pl.BlockSpec((BM, dout), lambda i: (jnp.maximum(i - 1, 0), 0)),
        out_shape=jax.ShapeDtypeStruct((N, dout), jnp.float32),
        scratch_shapes=[pltpu.VMEM((N, dout), jnp.float32)],
    )(adjacency, input_feature, W, b2, bo2)
